# Initial kernel scaffold; baseline (speedup 1.0000x reference)
import jax
import jax.numpy as jnp
from jax.experimental import pallas as pl


def kernel(x, edge_index, W1, b1, W2, b2, W3, b3, Wo, bo):
    raise NotImplementedError("write your pallas kernel here")


# speedup vs baseline: 20.8534x; 20.8534x over previous
import functools

import jax
import jax.numpy as jnp
from jax import lax
from jax.experimental import pallas as pl
from jax.experimental.pallas import tpu as pltpu
from jax.experimental.pallas import tpu_sc as plsc

N = 10000
E = 320000
IN_CH = 128
HID = 64

NC = 2
NS = 16
NW = NC * NS
EPW = E // NW
C = 80
NCH = EPW // C
NP = 10240
ZROWS = NP // NW

_mesh = plsc.VectorSubcoreMesh(core_axis_name="c", subcore_axis_name="s")


@functools.partial(
    pl.kernel,
    mesh=_mesh,
    out_type=jax.ShapeDtypeStruct((NC, NP, 16), jnp.float32),
    scratch_types=[
        pltpu.VMEM((NCH, C), jnp.int32),
        pltpu.VMEM((C, 16), jnp.float32),
        pltpu.VMEM_SHARED((NP, 16), jnp.float32),
    ],
)
def _deg_sc(dst_hbm, ones_hbm, z16_hbm, out_hbm, dstv, onesv, accd):
    cid = lax.axis_index("c")
    sid = lax.axis_index("s")
    wid = cid * NS + sid
    pltpu.sync_copy(dst_hbm.at[wid], dstv)
    pltpu.sync_copy(ones_hbm, onesv)
    pltpu.sync_copy(z16_hbm, accd.at[pl.ds(sid * ZROWS, ZROWS)])
    plsc.subcore_barrier()

    @pl.loop(0, NCH)
    def _(j):
        pltpu.sync_copy(onesv, accd.at[dstv.at[j]], add=True)

    plsc.subcore_barrier()
    pltpu.sync_copy(accd.at[pl.ds(sid * ZROWS, ZROWS)],
                    out_hbm.at[cid, pl.ds(sid * ZROWS, ZROWS)])


@functools.partial(
    pl.kernel,
    mesh=_mesh,
    compiler_params=pltpu.CompilerParams(use_tc_tiling_on_sc=False),
    out_type=jax.ShapeDtypeStruct((NC, NP, HID), jnp.float32),
    scratch_types=[
        pltpu.VMEM((NCH, C), jnp.int32),
        pltpu.VMEM((NCH, C), jnp.int32),
        pltpu.VMEM((C, HID), jnp.float32),
        pltpu.VMEM_SHARED((NP, HID), jnp.float32),
        pltpu.SemaphoreType.DMA,
    ],
)
def _agg_sc(g_hbm, src_hbm, dst_hbm, z64_hbm, out_hbm, srcv, dstv, rows, acc, sem):
    cid = lax.axis_index("c")
    sid = lax.axis_index("s")
    wid = cid * NS + sid
    pltpu.sync_copy(src_hbm.at[wid], srcv)
    pltpu.sync_copy(dst_hbm.at[wid], dstv)
    pltpu.sync_copy(z64_hbm, acc.at[pl.ds(sid * ZROWS, ZROWS)])
    plsc.subcore_barrier()

    @pl.loop(0, NCH)
    def _(j):
        pltpu.async_copy(g_hbm.at[srcv.at[j]], rows, sem).wait()
        pltpu.sync_copy(rows, acc.at[dstv.at[j]], add=True)

    plsc.subcore_barrier()
    pltpu.sync_copy(acc.at[pl.ds(sid * ZROWS, ZROWS)],
                    out_hbm.at[cid, pl.ds(sid * ZROWS, ZROWS)])


def _tc1_body(degp_ref, x_ref, w_ref, dinv_ref, g_ref):
    d = degp_ref[...]
    deg = 1.0 + d[0, :, 0:1] + d[1, :, 0:1]
    dinv = lax.rsqrt(deg)
    dinv_ref[...] = dinv
    h = jnp.dot(x_ref[...], w_ref[...], preferred_element_type=jnp.float32)
    g_ref[...] = h * dinv


def _tcmid_body(accp_ref, g_ref, dinv_ref, b_ref, w_ref, gout_ref):
    a = accp_ref[...]
    dinv = dinv_ref[...]
    y = jnp.maximum((a[0] + a[1] + g_ref[...]) * dinv + b_ref[...], 0.0)
    gout_ref[...] = jnp.dot(y, w_ref[...], preferred_element_type=jnp.float32) * dinv


def _tcfin_body(accp_ref, g_ref, dinv_ref, b_ref, wo_ref, bo_ref, out_ref):
    a = accp_ref[...]
    y = jnp.maximum((a[0] + a[1] + g_ref[...]) * dinv_ref[...] + b_ref[...], 0.0)
    out_ref[...] = jnp.dot(y, wo_ref[...], preferred_element_type=jnp.float32) + bo_ref[...]


def _tc1(degp, xp, W1):
    return pl.pallas_call(
        _tc1_body,
        out_shape=(jax.ShapeDtypeStruct((NP, 1), jnp.float32),
                   jax.ShapeDtypeStruct((NP, HID), jnp.float32)),
    )(degp, xp, W1)


def _tcmid(accp, g, dinv, b, W):
    return pl.pallas_call(
        _tcmid_body,
        out_shape=jax.ShapeDtypeStruct((NP, HID), jnp.float32),
    )(accp, g, dinv, b, W)


def _tcfin(accp, g, dinv, b, Wo, bo):
    return pl.pallas_call(
        _tcfin_body,
        out_shape=jax.ShapeDtypeStruct((NP, 1), jnp.float32),
    )(accp, g, dinv, b, Wo, bo)


def kernel(x, edge_index, W1, b1, W2, b2, W3, b3, Wo, bo):
    ei = edge_index.astype(jnp.int32)
    src = ei[0].reshape(NW, NCH, C)
    dst = ei[1].reshape(NW, NCH, C)
    xp = jnp.pad(x, ((0, NP - N), (0, 0)))
    z16 = jnp.zeros((ZROWS, 16), jnp.float32)
    z64 = jnp.zeros((ZROWS, HID), jnp.float32)
    ones16 = jnp.ones((C, 16), jnp.float32)

    degp = _deg_sc(dst, ones16, z16)
    dinv, g1 = _tc1(degp, xp, W1)
    acc1 = _agg_sc(g1, src, dst, z64)
    g2 = _tcmid(acc1, g1, dinv, b1.reshape(1, HID), W2)
    acc2 = _agg_sc(g2, src, dst, z64)
    g3 = _tcmid(acc2, g2, dinv, b2.reshape(1, HID), W3)
    acc3 = _agg_sc(g3, src, dst, z64)
    out = _tcfin(acc3, g3, dinv, b3.reshape(1, HID), Wo, bo.reshape(1, 1))
    return out[:N, 0]


# speedup vs baseline: 24.6803x; 1.1835x over previous
import functools

import jax
import jax.numpy as jnp
from jax import lax
from jax.experimental import pallas as pl
from jax.experimental.pallas import tpu as pltpu
from jax.experimental.pallas import tpu_sc as plsc

N = 10000
E = 320000
IN_CH = 128
HID = 64

NC = 2
NS = 16
NW = NC * NS
EPW = E // NW
C = 80
NCH = EPW // C
NP = 10240
ZROWS = NP // NW

_mesh = plsc.VectorSubcoreMesh(core_axis_name="c", subcore_axis_name="s")


@functools.partial(
    pl.kernel,
    mesh=_mesh,
    out_type=jax.ShapeDtypeStruct((NC, NP, 16), jnp.float32),
    scratch_types=[
        pltpu.VMEM((NCH, C), jnp.int32),
        pltpu.VMEM((C, 16), jnp.float32),
        pltpu.VMEM_SHARED((NP, 16), jnp.float32),
    ],
)
def _deg_sc(dst_hbm, ones_hbm, z16_hbm, out_hbm, dstv, onesv, accd):
    cid = lax.axis_index("c")
    sid = lax.axis_index("s")
    wid = cid * NS + sid
    pltpu.sync_copy(dst_hbm.at[wid], dstv)
    pltpu.sync_copy(ones_hbm, onesv)
    pltpu.sync_copy(z16_hbm, accd.at[pl.ds(sid * ZROWS, ZROWS)])
    plsc.subcore_barrier()

    @pl.loop(0, NCH)
    def _(j):
        pltpu.sync_copy(onesv, accd.at[dstv.at[j]], add=True)

    plsc.subcore_barrier()
    pltpu.sync_copy(accd.at[pl.ds(sid * ZROWS, ZROWS)],
                    out_hbm.at[cid, pl.ds(sid * ZROWS, ZROWS)])


@functools.partial(
    pl.kernel,
    mesh=_mesh,
    compiler_params=pltpu.CompilerParams(use_tc_tiling_on_sc=False),
    out_type=jax.ShapeDtypeStruct((NC, NP, HID), jnp.float32),
    scratch_types=[
        pltpu.VMEM((NCH, C), jnp.int32),
        pltpu.VMEM((NCH, C), jnp.int32),
        pltpu.VMEM((C, HID), jnp.float32),
        pltpu.VMEM((C, HID), jnp.float32),
        pltpu.VMEM_SHARED((NP, HID), jnp.float32),
        pltpu.SemaphoreType.DMA,
        pltpu.SemaphoreType.DMA,
    ],
)
def _agg_sc(g_hbm, src_hbm, dst_hbm, z64_hbm, out_hbm, srcv, dstv, rows_a, rows_b,
            acc, gsa, gsb):
    cid = lax.axis_index("c")
    sid = lax.axis_index("s")
    wid = cid * NS + sid
    pltpu.sync_copy(src_hbm.at[wid], srcv)
    pltpu.sync_copy(dst_hbm.at[wid], dstv)
    pltpu.sync_copy(z64_hbm, acc.at[pl.ds(sid * ZROWS, ZROWS)])
    plsc.subcore_barrier()

    def g_start(j, buf, sem):
        pltpu.async_copy(g_hbm.at[srcv.at[j]], buf, sem)

    def g_wait(j, buf, sem):
        pltpu.make_async_copy(g_hbm.at[srcv.at[j]], buf, sem).wait()

    g_start(0, rows_a, gsa)

    @pl.loop(0, NCH - 1, step=2)
    def _(j):
        g_wait(j, rows_a, gsa)
        g_start(j + 1, rows_b, gsb)
        pltpu.sync_copy(rows_a, acc.at[dstv.at[j]], add=True)
        g_wait(j + 1, rows_b, gsb)

        @pl.when(j + 2 < NCH)
        def _():
            g_start(j + 2, rows_a, gsa)

        pltpu.sync_copy(rows_b, acc.at[dstv.at[j + 1]], add=True)

    g_wait(NCH - 1, rows_a, gsa)
    pltpu.sync_copy(rows_a, acc.at[dstv.at[NCH - 1]], add=True)
    plsc.subcore_barrier()
    pltpu.sync_copy(acc.at[pl.ds(sid * ZROWS, ZROWS)],
                    out_hbm.at[cid, pl.ds(sid * ZROWS, ZROWS)])


def _tc1_body(degp_ref, x_ref, w_ref, dinv_ref, g_ref):
    d = degp_ref[...]
    deg = 1.0 + d[0, :, 0:1] + d[1, :, 0:1]
    dinv = lax.rsqrt(deg)
    dinv_ref[...] = dinv
    h = jnp.dot(x_ref[...], w_ref[...], preferred_element_type=jnp.float32)
    g_ref[...] = h * dinv


def _tcmid_body(accp_ref, g_ref, dinv_ref, b_ref, w_ref, gout_ref):
    a = accp_ref[...]
    dinv = dinv_ref[...]
    y = jnp.maximum((a[0] + a[1] + g_ref[...]) * dinv + b_ref[...], 0.0)
    gout_ref[...] = jnp.dot(y, w_ref[...], preferred_element_type=jnp.float32) * dinv


def _tcfin_body(accp_ref, g_ref, dinv_ref, b_ref, wo_ref, bo_ref, out_ref):
    a = accp_ref[...]
    y = jnp.maximum((a[0] + a[1] + g_ref[...]) * dinv_ref[...] + b_ref[...], 0.0)
    out_ref[...] = jnp.dot(y, wo_ref[...], preferred_element_type=jnp.float32) + bo_ref[...]


def _tc1(degp, xp, W1):
    return pl.pallas_call(
        _tc1_body,
        out_shape=(jax.ShapeDtypeStruct((NP, 1), jnp.float32),
                   jax.ShapeDtypeStruct((NP, HID), jnp.float32)),
    )(degp, xp, W1)


def _tcmid(accp, g, dinv, b, W):
    return pl.pallas_call(
        _tcmid_body,
        out_shape=jax.ShapeDtypeStruct((NP, HID), jnp.float32),
    )(accp, g, dinv, b, W)


def _tcfin(accp, g, dinv, b, Wo, bo):
    return pl.pallas_call(
        _tcfin_body,
        out_shape=jax.ShapeDtypeStruct((NP, 1), jnp.float32),
    )(accp, g, dinv, b, Wo, bo)


def kernel(x, edge_index, W1, b1, W2, b2, W3, b3, Wo, bo):
    ei = edge_index.astype(jnp.int32)
    src = ei[0].reshape(NW, NCH, C)
    dst = ei[1].reshape(NW, NCH, C)
    xp = jnp.pad(x, ((0, NP - N), (0, 0)))
    z16 = jnp.zeros((ZROWS, 16), jnp.float32)
    z64 = jnp.zeros((ZROWS, HID), jnp.float32)
    ones16 = jnp.ones((C, 16), jnp.float32)

    degp = _deg_sc(dst, ones16, z16)
    dinv, g1 = _tc1(degp, xp, W1)
    acc1 = _agg_sc(g1, src, dst, z64)
    g2 = _tcmid(acc1, g1, dinv, b1.reshape(1, HID), W2)
    acc2 = _agg_sc(g2, src, dst, z64)
    g3 = _tcmid(acc2, g2, dinv, b2.reshape(1, HID), W3)
    acc3 = _agg_sc(g3, src, dst, z64)
    out = _tcfin(acc3, g3, dinv, b3.reshape(1, HID), Wo, bo.reshape(1, 1))
    return out[:N, 0]


# speedup vs baseline: 31.6644x; 1.2830x over previous
import functools

import jax
import jax.numpy as jnp
from jax import lax
from jax.experimental import pallas as pl
from jax.experimental.pallas import tpu as pltpu
from jax.experimental.pallas import tpu_sc as plsc

N = 10000
E = 320000
IN_CH = 128
HID = 64

NC = 2
NS = 16
NW = NC * NS
EPW = E // NW
C = 80
NCH = EPW // C
NP = 10240
ZROWS = NP // NW

_mesh = plsc.VectorSubcoreMesh(core_axis_name="c", subcore_axis_name="s")


@functools.partial(
    pl.kernel,
    mesh=_mesh,
    out_type=jax.ShapeDtypeStruct((NC, NP, 16), jnp.float32),
    scratch_types=[
        pltpu.VMEM((NCH, C), jnp.int32),
        pltpu.VMEM((C, 16), jnp.float32),
        pltpu.VMEM_SHARED((NP, 16), jnp.float32),
    ],
)
def _deg_sc(dst_hbm, ones_hbm, z16_hbm, out_hbm, dstv, onesv, accd):
    cid = lax.axis_index("c")
    sid = lax.axis_index("s")
    wid = cid * NS + sid
    pltpu.sync_copy(dst_hbm.at[wid], dstv)
    pltpu.sync_copy(ones_hbm, onesv)
    pltpu.sync_copy(z16_hbm, accd.at[pl.ds(sid * ZROWS, ZROWS)])
    plsc.subcore_barrier()

    @pl.loop(0, NCH)
    def _(j):
        pltpu.sync_copy(onesv, accd.at[dstv.at[j]], add=True)

    plsc.subcore_barrier()
    pltpu.sync_copy(accd.at[pl.ds(sid * ZROWS, ZROWS)],
                    out_hbm.at[cid, pl.ds(sid * ZROWS, ZROWS)])


@functools.partial(
    pl.kernel,
    mesh=_mesh,
    compiler_params=pltpu.CompilerParams(use_tc_tiling_on_sc=False),
    out_type=jax.ShapeDtypeStruct((NC, NP, HID), jnp.float32),
    scratch_types=[
        pltpu.VMEM((NCH, C), jnp.int32),
        pltpu.VMEM((NCH, C), jnp.int32),
        pltpu.VMEM((C, HID), jnp.float32),
        pltpu.VMEM((C, HID), jnp.float32),
        pltpu.VMEM_SHARED((NP, HID), jnp.float32),
        pltpu.VMEM_SHARED((NP, HID), jnp.float32),
        pltpu.SemaphoreType.DMA,
        pltpu.SemaphoreType.DMA,
    ],
)
def _agg_sc(g_hbm, src_hbm, dst_hbm, z64_hbm, out_hbm, srcv, dstv, rows_a, rows_b,
            acc, gtab, gsa, gsb):
    cid = lax.axis_index("c")
    sid = lax.axis_index("s")
    wid = cid * NS + sid
    pltpu.sync_copy(src_hbm.at[wid], srcv)
    pltpu.sync_copy(dst_hbm.at[wid], dstv)
    pltpu.sync_copy(z64_hbm, acc.at[pl.ds(sid * ZROWS, ZROWS)])
    pltpu.sync_copy(g_hbm.at[pl.ds(sid * ZROWS, ZROWS)],
                    gtab.at[pl.ds(sid * ZROWS, ZROWS)])
    plsc.subcore_barrier()

    def g_start(j, buf, sem):
        pltpu.async_copy(gtab.at[srcv.at[j]], buf, sem)

    def g_wait(j, buf, sem):
        pltpu.make_async_copy(gtab.at[srcv.at[j]], buf, sem).wait()

    g_start(0, rows_a, gsa)

    @pl.loop(0, NCH - 1, step=2)
    def _(j):
        g_wait(j, rows_a, gsa)
        g_start(j + 1, rows_b, gsb)
        pltpu.sync_copy(rows_a, acc.at[dstv.at[j]], add=True)
        g_wait(j + 1, rows_b, gsb)

        @pl.when(j + 2 < NCH)
        def _():
            g_start(j + 2, rows_a, gsa)

        pltpu.sync_copy(rows_b, acc.at[dstv.at[j + 1]], add=True)

    g_wait(NCH - 1, rows_a, gsa)
    pltpu.sync_copy(rows_a, acc.at[dstv.at[NCH - 1]], add=True)
    plsc.subcore_barrier()
    pltpu.sync_copy(acc.at[pl.ds(sid * ZROWS, ZROWS)],
                    out_hbm.at[cid, pl.ds(sid * ZROWS, ZROWS)])


def _tc1_body(degp_ref, x_ref, w_ref, dinv_ref, g_ref):
    d = degp_ref[...]
    deg = 1.0 + d[0, :, 0:1] + d[1, :, 0:1]
    dinv = lax.rsqrt(deg)
    dinv_ref[...] = dinv
    h = jnp.dot(x_ref[...], w_ref[...], preferred_element_type=jnp.float32)
    g_ref[...] = h * dinv


def _tcmid_body(accp_ref, g_ref, dinv_ref, b_ref, w_ref, gout_ref):
    a = accp_ref[...]
    dinv = dinv_ref[...]
    y = jnp.maximum((a[0] + a[1] + g_ref[...]) * dinv + b_ref[...], 0.0)
    gout_ref[...] = jnp.dot(y, w_ref[...], preferred_element_type=jnp.float32) * dinv


def _tcfin_body(accp_ref, g_ref, dinv_ref, b_ref, wo_ref, bo_ref, out_ref):
    a = accp_ref[...]
    y = jnp.maximum((a[0] + a[1] + g_ref[...]) * dinv_ref[...] + b_ref[...], 0.0)
    out_ref[...] = jnp.dot(y, wo_ref[...], preferred_element_type=jnp.float32) + bo_ref[...]


def _tc1(degp, xp, W1):
    return pl.pallas_call(
        _tc1_body,
        out_shape=(jax.ShapeDtypeStruct((NP, 1), jnp.float32),
                   jax.ShapeDtypeStruct((NP, HID), jnp.float32)),
    )(degp, xp, W1)


def _tcmid(accp, g, dinv, b, W):
    return pl.pallas_call(
        _tcmid_body,
        out_shape=jax.ShapeDtypeStruct((NP, HID), jnp.float32),
    )(accp, g, dinv, b, W)


def _tcfin(accp, g, dinv, b, Wo, bo):
    return pl.pallas_call(
        _tcfin_body,
        out_shape=jax.ShapeDtypeStruct((NP, 1), jnp.float32),
    )(accp, g, dinv, b, Wo, bo)


def kernel(x, edge_index, W1, b1, W2, b2, W3, b3, Wo, bo):
    ei = edge_index.astype(jnp.int32)
    src = ei[0].reshape(NW, NCH, C)
    dst = ei[1].reshape(NW, NCH, C)
    xp = jnp.pad(x, ((0, NP - N), (0, 0)))
    z16 = jnp.zeros((ZROWS, 16), jnp.float32)
    z64 = jnp.zeros((ZROWS, HID), jnp.float32)
    ones16 = jnp.ones((C, 16), jnp.float32)

    degp = _deg_sc(dst, ones16, z16)
    dinv, g1 = _tc1(degp, xp, W1)
    acc1 = _agg_sc(g1, src, dst, z64)
    g2 = _tcmid(acc1, g1, dinv, b1.reshape(1, HID), W2)
    acc2 = _agg_sc(g2, src, dst, z64)
    g3 = _tcmid(acc2, g2, dinv, b2.reshape(1, HID), W3)
    acc3 = _agg_sc(g3, src, dst, z64)
    out = _tcfin(acc3, g3, dinv, b3.reshape(1, HID), Wo, bo.reshape(1, 1))
    return out[:N, 0]


# speedup vs baseline: 37.6594x; 1.1893x over previous
import functools

import jax
import jax.numpy as jnp
from jax import lax
from jax.experimental import pallas as pl
from jax.experimental.pallas import tpu as pltpu
from jax.experimental.pallas import tpu_sc as plsc

N = 10000
E = 320000
IN_CH = 128
HID = 64

NC = 2
NS = 16
NW = NC * NS
EPW = E // NW
C = 80
NCH = EPW // C
NP = 10240
ZROWS = NP // NW

_mesh = plsc.VectorSubcoreMesh(core_axis_name="c", subcore_axis_name="s")


@functools.partial(
    pl.kernel,
    mesh=_mesh,
    out_type=jax.ShapeDtypeStruct((NC, NP, 16), jnp.float32),
    scratch_types=[
        pltpu.VMEM((NCH, C), jnp.int32),
        pltpu.VMEM((C, 16), jnp.float32),
        pltpu.VMEM_SHARED((NP, 16), jnp.float32),
    ],
)
def _deg_sc(dst_hbm, ones_hbm, z16_hbm, out_hbm, dstv, onesv, accd):
    cid = lax.axis_index("c")
    sid = lax.axis_index("s")
    wid = cid * NS + sid
    pltpu.sync_copy(dst_hbm.at[wid], dstv)
    pltpu.sync_copy(ones_hbm, onesv)
    pltpu.sync_copy(z16_hbm, accd.at[pl.ds(sid * ZROWS, ZROWS)])
    plsc.subcore_barrier()

    @pl.loop(0, NCH)
    def _(j):
        pltpu.sync_copy(onesv, accd.at[dstv.at[j]], add=True)

    plsc.subcore_barrier()
    pltpu.sync_copy(accd.at[pl.ds(sid * ZROWS, ZROWS)],
                    out_hbm.at[cid, pl.ds(sid * ZROWS, ZROWS)])


NBUF = 10
MAIN = NCH - NBUF // 2


@functools.partial(
    pl.kernel,
    mesh=_mesh,
    compiler_params=pltpu.CompilerParams(use_tc_tiling_on_sc=False),
    out_type=jax.ShapeDtypeStruct((NC, NP, HID), jnp.float32),
    scratch_types=(
        [pltpu.VMEM((NCH, C), jnp.int32),
         pltpu.VMEM((NCH, C), jnp.int32)]
        + [pltpu.VMEM((C, HID), jnp.float32)] * NBUF
        + [pltpu.VMEM_SHARED((NP, HID), jnp.float32)]
        + [pltpu.SemaphoreType.DMA] * NBUF
    ),
)
def _agg_sc(g_hbm, src_hbm, dst_hbm, z64_hbm, out_hbm, srcv, dstv, *rest):
    bufs = rest[:NBUF]
    acc = rest[NBUF]
    sems = rest[NBUF + 1:]
    cid = lax.axis_index("c")
    sid = lax.axis_index("s")
    wid = cid * NS + sid
    pltpu.sync_copy(src_hbm.at[wid], srcv)
    pltpu.sync_copy(dst_hbm.at[wid], dstv)
    pltpu.sync_copy(z64_hbm, acc.at[pl.ds(sid * ZROWS, ZROWS)])
    plsc.subcore_barrier()

    def g_start(j, buf, sem):
        pltpu.async_copy(g_hbm.at[srcv.at[j]], buf, sem)

    def g_wait(j, buf, sem):
        pltpu.make_async_copy(g_hbm.at[srcv.at[j]], buf, sem).wait()

    for b in range(NBUF):
        g_start(b, bufs[b], sems[b])

    @pl.loop(0, MAIN, step=NBUF)
    def _(j0):
        hs = []
        for b in range(NBUF):
            g_wait(j0 + b, bufs[b], sems[b])
            hs.append(pltpu.async_copy(bufs[b], acc.at[dstv.at[j0 + b]],
                                       sems[b], add=True))
        for b in range(NBUF):
            hs[b].wait()

            @pl.when(j0 + b + NBUF < NCH)
            def _():
                g_start(j0 + b + NBUF, bufs[b], sems[b])

    for b in range(NCH - MAIN):
        g_wait(MAIN + b, bufs[b], sems[b])
        pltpu.sync_copy(bufs[b], acc.at[dstv.at[MAIN + b]], add=True)
    plsc.subcore_barrier()
    pltpu.sync_copy(acc.at[pl.ds(sid * ZROWS, ZROWS)],
                    out_hbm.at[cid, pl.ds(sid * ZROWS, ZROWS)])


def _tc1_body(degp_ref, x_ref, w_ref, dinv_ref, g_ref):
    d = degp_ref[...]
    deg = 1.0 + d[0, :, 0:1] + d[1, :, 0:1]
    dinv = lax.rsqrt(deg)
    dinv_ref[...] = dinv
    h = jnp.dot(x_ref[...], w_ref[...], preferred_element_type=jnp.float32)
    g_ref[...] = h * dinv


def _tcmid_body(accp_ref, g_ref, dinv_ref, b_ref, w_ref, gout_ref):
    a = accp_ref[...]
    dinv = dinv_ref[...]
    y = jnp.maximum((a[0] + a[1] + g_ref[...]) * dinv + b_ref[...], 0.0)
    gout_ref[...] = jnp.dot(y, w_ref[...], preferred_element_type=jnp.float32) * dinv


def _tcfin_body(accp_ref, g_ref, dinv_ref, b_ref, wo_ref, bo_ref, out_ref):
    a = accp_ref[...]
    y = jnp.maximum((a[0] + a[1] + g_ref[...]) * dinv_ref[...] + b_ref[...], 0.0)
    out_ref[...] = jnp.dot(y, wo_ref[...], preferred_element_type=jnp.float32) + bo_ref[...]


def _tc1(degp, xp, W1):
    return pl.pallas_call(
        _tc1_body,
        out_shape=(jax.ShapeDtypeStruct((NP, 1), jnp.float32),
                   jax.ShapeDtypeStruct((NP, HID), jnp.float32)),
    )(degp, xp, W1)


def _tcmid(accp, g, dinv, b, W):
    return pl.pallas_call(
        _tcmid_body,
        out_shape=jax.ShapeDtypeStruct((NP, HID), jnp.float32),
    )(accp, g, dinv, b, W)


def _tcfin(accp, g, dinv, b, Wo, bo):
    return pl.pallas_call(
        _tcfin_body,
        out_shape=jax.ShapeDtypeStruct((NP, 1), jnp.float32),
    )(accp, g, dinv, b, Wo, bo)


def kernel(x, edge_index, W1, b1, W2, b2, W3, b3, Wo, bo):
    ei = edge_index.astype(jnp.int32)
    src = ei[0].reshape(NW, NCH, C)
    dst = ei[1].reshape(NW, NCH, C)
    xp = jnp.pad(x, ((0, NP - N), (0, 0)))
    z16 = jnp.zeros((ZROWS, 16), jnp.float32)
    z64 = jnp.zeros((ZROWS, HID), jnp.float32)
    ones16 = jnp.ones((C, 16), jnp.float32)

    degp = _deg_sc(dst, ones16, z16)
    dinv, g1 = _tc1(degp, xp, W1)
    acc1 = _agg_sc(g1, src, dst, z64)
    g2 = _tcmid(acc1, g1, dinv, b1.reshape(1, HID), W2)
    acc2 = _agg_sc(g2, src, dst, z64)
    g3 = _tcmid(acc2, g2, dinv, b2.reshape(1, HID), W3)
    acc3 = _agg_sc(g3, src, dst, z64)
    out = _tcfin(acc3, g3, dinv, b3.reshape(1, HID), Wo, bo.reshape(1, 1))
    return out[:N, 0]


# speedup vs baseline: 40.0315x; 1.0630x over previous
import functools

import jax
import jax.numpy as jnp
from jax import lax
from jax.experimental import pallas as pl
from jax.experimental.pallas import tpu as pltpu
from jax.experimental.pallas import tpu_sc as plsc

N = 10000
E = 320000
IN_CH = 128
HID = 64

NC = 2
NS = 16
NW = NC * NS
EPW = E // NW
C = 80
NCH = EPW // C
NP = 10240
ZROWS = NP // NW

_mesh = plsc.VectorSubcoreMesh(core_axis_name="c", subcore_axis_name="s")


@functools.partial(
    pl.kernel,
    mesh=_mesh,
    out_type=jax.ShapeDtypeStruct((NC, NP, 16), jnp.float32),
    scratch_types=[
        pltpu.VMEM((NCH, C), jnp.int32),
        pltpu.VMEM((C, 16), jnp.float32),
        pltpu.VMEM_SHARED((NP, 16), jnp.float32),
    ],
)
def _deg_sc(dst_hbm, ones_hbm, z16_hbm, out_hbm, dstv, onesv, accd):
    cid = lax.axis_index("c")
    sid = lax.axis_index("s")
    wid = cid * NS + sid
    pltpu.sync_copy(dst_hbm.at[wid], dstv)
    pltpu.sync_copy(ones_hbm, onesv)
    pltpu.sync_copy(z16_hbm, accd.at[pl.ds(sid * ZROWS, ZROWS)])
    plsc.subcore_barrier()

    @pl.loop(0, NCH)
    def _(j):
        pltpu.sync_copy(onesv, accd.at[dstv.at[j]], add=True)

    plsc.subcore_barrier()
    pltpu.sync_copy(accd.at[pl.ds(sid * ZROWS, ZROWS)],
                    out_hbm.at[cid, pl.ds(sid * ZROWS, ZROWS)])


NBUF = 10
MAIN = NCH - NBUF // 2


@functools.partial(
    pl.kernel,
    mesh=_mesh,
    compiler_params=pltpu.CompilerParams(use_tc_tiling_on_sc=False),
    out_type=jax.ShapeDtypeStruct((NC, NP, HID), jnp.float32),
    scratch_types=(
        [pltpu.VMEM((NCH, C), jnp.int32),
         pltpu.VMEM((NCH, C), jnp.int32)]
        + [pltpu.VMEM((C, HID), jnp.float32)] * NBUF
        + [pltpu.VMEM_SHARED((NP, HID), jnp.float32)]
        + [pltpu.SemaphoreType.DMA] * NBUF
    ),
)
def _agg_sc(g_hbm, src_hbm, dst_hbm, z64_hbm, out_hbm, srcv, dstv, *rest):
    bufs = rest[:NBUF]
    acc = rest[NBUF]
    sems = rest[NBUF + 1:]
    cid = lax.axis_index("c")
    sid = lax.axis_index("s")
    wid = cid * NS + sid
    pltpu.sync_copy(src_hbm.at[wid], srcv)
    pltpu.sync_copy(dst_hbm.at[wid], dstv)
    pltpu.sync_copy(z64_hbm, acc.at[pl.ds(sid * ZROWS, ZROWS)])
    plsc.subcore_barrier()

    def g_start(j, buf, sem):
        pltpu.async_copy(g_hbm.at[srcv.at[j]], buf, sem)

    def g_wait(j, buf, sem):
        pltpu.make_async_copy(g_hbm.at[srcv.at[j]], buf, sem).wait()

    for b in range(NBUF):
        g_start(b, bufs[b], sems[b])

    @pl.loop(0, MAIN, step=NBUF)
    def _(j0):
        for b in range(NBUF):
            g_wait(j0 + b, bufs[b], sems[b])
            pltpu.sync_copy(bufs[b], acc.at[dstv.at[j0 + b]], add=True)

            @pl.when(j0 + b + NBUF < NCH)
            def _():
                g_start(j0 + b + NBUF, bufs[b], sems[b])

    for b in range(NCH - MAIN):
        g_wait(MAIN + b, bufs[b], sems[b])
        pltpu.sync_copy(bufs[b], acc.at[dstv.at[MAIN + b]], add=True)
    plsc.subcore_barrier()
    pltpu.sync_copy(acc.at[pl.ds(sid * ZROWS, ZROWS)],
                    out_hbm.at[cid, pl.ds(sid * ZROWS, ZROWS)])


def _tc1_body(degp_ref, x_ref, w_ref, dinv_ref, g_ref):
    d = degp_ref[...]
    deg = 1.0 + d[0, :, 0:1] + d[1, :, 0:1]
    dinv = lax.rsqrt(deg)
    dinv_ref[...] = dinv
    h = jnp.dot(x_ref[...], w_ref[...], preferred_element_type=jnp.float32)
    g_ref[...] = h * dinv


def _tcmid_body(accp_ref, g_ref, dinv_ref, b_ref, w_ref, gout_ref):
    a = accp_ref[...]
    dinv = dinv_ref[...]
    y = jnp.maximum((a[0] + a[1] + g_ref[...]) * dinv + b_ref[...], 0.0)
    gout_ref[...] = jnp.dot(y, w_ref[...], preferred_element_type=jnp.float32) * dinv


def _tcfin_body(accp_ref, g_ref, dinv_ref, b_ref, wo_ref, bo_ref, out_ref):
    a = accp_ref[...]
    y = jnp.maximum((a[0] + a[1] + g_ref[...]) * dinv_ref[...] + b_ref[...], 0.0)
    out_ref[...] = jnp.dot(y, wo_ref[...], preferred_element_type=jnp.float32) + bo_ref[...]


def _tc1(degp, xp, W1):
    return pl.pallas_call(
        _tc1_body,
        out_shape=(jax.ShapeDtypeStruct((NP, 1), jnp.float32),
                   jax.ShapeDtypeStruct((NP, HID), jnp.float32)),
    )(degp, xp, W1)


def _tcmid(accp, g, dinv, b, W):
    return pl.pallas_call(
        _tcmid_body,
        out_shape=jax.ShapeDtypeStruct((NP, HID), jnp.float32),
    )(accp, g, dinv, b, W)


def _tcfin(accp, g, dinv, b, Wo, bo):
    return pl.pallas_call(
        _tcfin_body,
        out_shape=jax.ShapeDtypeStruct((NP, 1), jnp.float32),
    )(accp, g, dinv, b, Wo, bo)


def kernel(x, edge_index, W1, b1, W2, b2, W3, b3, Wo, bo):
    ei = edge_index.astype(jnp.int32)
    src = ei[0].reshape(NW, NCH, C)
    dst = ei[1].reshape(NW, NCH, C)
    xp = jnp.pad(x, ((0, NP - N), (0, 0)))
    z16 = jnp.zeros((ZROWS, 16), jnp.float32)
    z64 = jnp.zeros((ZROWS, HID), jnp.float32)
    ones16 = jnp.ones((C, 16), jnp.float32)

    degp = _deg_sc(dst, ones16, z16)
    dinv, g1 = _tc1(degp, xp, W1)
    acc1 = _agg_sc(g1, src, dst, z64)
    g2 = _tcmid(acc1, g1, dinv, b1.reshape(1, HID), W2)
    acc2 = _agg_sc(g2, src, dst, z64)
    g3 = _tcmid(acc2, g2, dinv, b2.reshape(1, HID), W3)
    acc3 = _agg_sc(g3, src, dst, z64)
    out = _tcfin(acc3, g3, dinv, b3.reshape(1, HID), Wo, bo.reshape(1, 1))
    return out[:N, 0]
